# Initial kernel scaffold; baseline (speedup 1.0000x reference)
import jax
import jax.numpy as jnp
from jax.experimental import pallas as pl


def kernel(word_sequences, table):
    raise NotImplementedError("write your pallas kernel here")


# speedup vs baseline: 1.5564x; 1.5564x over previous
import functools

import jax
import jax.numpy as jnp
from jax import lax
from jax.experimental import pallas as pl
from jax.experimental.pallas import tpu as pltpu
from jax.experimental.pallas import tpu_sc as plsc

VOCAB_SIZE = 1000000
EMBED_DIM = 32

NUM_CORES = 2
NUM_SUBCORES = 16
NUM_WORKERS = NUM_CORES * NUM_SUBCORES

B_TOTAL = 4096 * 200
B_PER_W = B_TOTAL // NUM_WORKERS
IDX_MINOR = 128
ROWS_PER_W = B_PER_W // IDX_MINOR
STREAMS_PER_CHUNK = 8
CHUNK = STREAMS_PER_CHUNK * IDX_MINOR
N_CHUNKS = B_PER_W // CHUNK


def _gather_body(idx_hbm, table_hbm, out_hbm, idx_v, rows_v, gsem):
    wid = lax.axis_index("s") * NUM_CORES + lax.axis_index("c")
    pltpu.sync_copy(idx_hbm.at[pl.ds(wid * ROWS_PER_W, ROWS_PER_W)], idx_v)

    out_base = wid * B_PER_W

    def chunk_body(c, carry):
        j0 = c * STREAMS_PER_CHUNK
        copies = []
        for s in range(STREAMS_PER_CHUNK):
            copies.append(
                pltpu.async_copy(
                    table_hbm.at[idx_v.at[j0 + s]],
                    rows_v.at[pl.ds(s * IDX_MINOR, IDX_MINOR)],
                    gsem,
                )
            )
        for cp in copies:
            cp.wait()
        pltpu.sync_copy(rows_v, out_hbm.at[pl.ds(out_base + c * CHUNK, CHUNK)])
        return carry

    lax.fori_loop(0, N_CHUNKS, chunk_body, 0)


@jax.jit
def _embed_gather(word_flat_2d, table):
    mesh = plsc.VectorSubcoreMesh(
        core_axis_name="c",
        subcore_axis_name="s",
        num_cores=NUM_CORES,
        num_subcores=NUM_SUBCORES,
    )
    return pl.kernel(
        _gather_body,
        out_type=jax.ShapeDtypeStruct((B_TOTAL, EMBED_DIM), jnp.float32),
        mesh=mesh,
        scratch_types=[
            pltpu.VMEM((ROWS_PER_W, IDX_MINOR), jnp.int32),
            pltpu.VMEM((CHUNK, EMBED_DIM), jnp.float32),
            pltpu.SemaphoreType.DMA,
        ],
        compiler_params=pltpu.CompilerParams(use_tc_tiling_on_sc=False),
    )(word_flat_2d, table)


def kernel(word_sequences, table):
    n, l = word_sequences.shape
    idx = word_sequences.astype(jnp.int32).reshape(B_TOTAL // IDX_MINOR, IDX_MINOR)
    out = _embed_gather(idx, table)
    return out.reshape(n, l, EMBED_DIM)


# speedup vs baseline: 1.5828x; 1.0170x over previous
import jax
import jax.numpy as jnp
from jax import lax
from jax.experimental import pallas as pl
from jax.experimental.pallas import tpu as pltpu
from jax.experimental.pallas import tpu_sc as plsc

VOCAB_SIZE = 1000000
EMBED_DIM = 32

NUM_CORES = 2
NUM_SUBCORES = 16
NUM_WORKERS = NUM_CORES * NUM_SUBCORES

B_TOTAL = 4096 * 200
B_PER_W = B_TOTAL // NUM_WORKERS
IDX_MINOR = 128
ROWS_PER_W = B_PER_W // IDX_MINOR
STREAMS_PER_CHUNK = 5
CHUNK = STREAMS_PER_CHUNK * IDX_MINOR
N_CHUNKS = B_PER_W // CHUNK
N_BUF = 4
CHUNK_BYTES = CHUNK * EMBED_DIM * 4


def _gather_body(idx_hbm, table_hbm, out_hbm, idx_v,
                 buf0, buf1, buf2, buf3,
                 g0, g1, g2, g3, s0, s1, s2, s3):
    bufs = [buf0, buf1, buf2, buf3]
    gsems = [g0, g1, g2, g3]
    ssems = [s0, s1, s2, s3]

    wid = lax.axis_index("s") * NUM_CORES + lax.axis_index("c")
    pltpu.sync_copy(idx_hbm.at[pl.ds(wid * ROWS_PER_W, ROWS_PER_W)], idx_v)
    out_base = wid * B_PER_W

    def fire_gathers(c, b):
        j0 = c * STREAMS_PER_CHUNK
        for t in range(STREAMS_PER_CHUNK):
            pltpu.async_copy(
                table_hbm.at[idx_v.at[j0 + t]],
                bufs[b].at[pl.ds(t * IDX_MINOR, IDX_MINOR)],
                gsems[b],
            )

    def fire_store(c, b):
        pltpu.async_copy(
            bufs[b], out_hbm.at[pl.ds(out_base + c * CHUNK, CHUNK)], ssems[b]
        )

    def drain(sem, b):
        pltpu.make_async_copy(out_hbm.at[pl.ds(0, CHUNK)], bufs[b], sem).wait()

    for c in range(N_BUF - 1):
        fire_gathers(c, c)
    drain(gsems[0], 0)
    fire_store(0, 0)
    fire_gathers(3, 3)

    def main_body(i, carry):
        for off in range(N_BUF):
            s = N_BUF * i + 1 + off
            b = (1 + off) % N_BUF
            bn = (b + 3) % N_BUF
            drain(ssems[bn], bn)
            fire_gathers(s + 3, bn)
            drain(gsems[b], b)
            fire_store(s, b)
        return carry

    lax.fori_loop(0, (N_CHUNKS - N_BUF) // N_BUF, main_body, 0)

    for s in range(N_CHUNKS - 3, N_CHUNKS):
        b = s % N_BUF
        drain(gsems[b], b)
        fire_store(s, b)
    for b in range(N_BUF):
        drain(ssems[b], b)


@jax.jit
def _embed_gather(word_flat_2d, table):
    mesh = plsc.VectorSubcoreMesh(
        core_axis_name="c",
        subcore_axis_name="s",
        num_cores=NUM_CORES,
        num_subcores=NUM_SUBCORES,
    )
    return pl.kernel(
        _gather_body,
        out_type=jax.ShapeDtypeStruct((B_TOTAL, EMBED_DIM), jnp.float32),
        mesh=mesh,
        scratch_types=(
            [pltpu.VMEM((ROWS_PER_W, IDX_MINOR), jnp.int32)]
            + [pltpu.VMEM((CHUNK, EMBED_DIM), jnp.float32) for _ in range(N_BUF)]
            + [pltpu.SemaphoreType.DMA for _ in range(2 * N_BUF)]
        ),
        compiler_params=pltpu.CompilerParams(use_tc_tiling_on_sc=False),
    )(word_flat_2d, table)


def kernel(word_sequences, table):
    n, l = word_sequences.shape
    idx = word_sequences.astype(jnp.int32).reshape(B_TOTAL // IDX_MINOR, IDX_MINOR)
    out = _embed_gather(idx, table)
    return out.reshape(n, l, EMBED_DIM)
